# baseline (speedup 1.0000x reference)
import functools

import jax
import jax.numpy as jnp
from jax import lax
from jax.experimental import pallas as pl
from jax.experimental.pallas import tpu as pltpu
from jax.experimental.pallas import tpu_sc as plsc

N, D, K = 8192, 32, 512
BN = 1024
NC, NS, L = 2, 16, 16
NW = NC * NS
PPW = N // NW
KB = K // L
CH = 16
NCH = PPW // CH

_mesh = plsc.VectorSubcoreMesh(core_axis_name="c", subcore_axis_name="s",
                               num_cores=NC, num_subcores=NS)


def _scores_body(x_ref, ct_ref, s_ref):
    x = x_ref[...]
    ct = ct_ref[...]
    cn = jnp.sum(ct * ct, axis=0)[None, :]
    dot = lax.dot_general(x, ct, (((1,), (0,)), ((), ())),
                          precision=lax.Precision.HIGHEST,
                          preferred_element_type=jnp.float32)
    s_ref[...] = cn - 2.0 * dot


@functools.partial(
    pl.kernel,
    out_type=jax.ShapeDtypeStruct((N,), jnp.int32),
    mesh=_mesh,
    scratch_types=[
        pltpu.VMEM((CH, K), jnp.float32),
        pltpu.VMEM((CH, K), jnp.float32),
        pltpu.VMEM((PPW,), jnp.int32),
        pltpu.SemaphoreType.DMA,
        pltpu.SemaphoreType.DMA,
    ],
    compiler_params=pltpu.CompilerParams(needs_layout_passes=False),
)
def _sc_argmin(s_hbm, out_hbm, buf0, buf1, out_v, sem0, sem1):
    wid = lax.axis_index("s") * NC + lax.axis_index("c")
    base = wid * PPW
    bufs = (buf0, buf1)
    sems = (sem0, sem1)
    iota = lax.iota(jnp.int32, L)
    lane0 = iota == 0

    cps = [None] * NCH
    cps[0] = pltpu.async_copy(s_hbm.at[pl.ds(base, CH)], buf0, sem0)
    for ch in range(NCH):
        if ch + 1 < NCH:
            cps[ch + 1] = pltpu.async_copy(
                s_hbm.at[pl.ds(base + (ch + 1) * CH, CH)],
                bufs[(ch + 1) % 2], sems[(ch + 1) % 2])
        cps[ch].wait()
        buf = bufs[ch % 2]
        rb = ch * CH

        def row_body(r, _, buf=buf, rb=rb):
            sv = [buf[r, pl.ds(kb * L, L)] for kb in range(KB)]
            bestv = sv[0]
            besti = jnp.zeros((L,), jnp.int32)
            for kb in range(1, KB):
                better = sv[kb] < bestv
                bestv = jnp.where(better, sv[kb], bestv)
                besti = jnp.where(better, jnp.full((L,), kb, jnp.int32),
                                  besti)
            m = jnp.min(bestv)
            cand = jnp.where(bestv == m, besti * L + iota, K)
            ridx = jnp.min(cand)
            plsc.store_scatter(out_v, [jnp.full((L,), rb, jnp.int32) + r],
                               jnp.full((L,), 0, jnp.int32) + ridx,
                               mask=lane0)
            return 0

        lax.fori_loop(0, CH, row_body, 0)
    pltpu.sync_copy(out_v, out_hbm.at[pl.ds(base, PPW)])


def kernel(x, centroids):
    scores = pl.pallas_call(
        _scores_body,
        grid=(N // BN,),
        in_specs=[
            pl.BlockSpec((BN, D), lambda i: (i, 0)),
            pl.BlockSpec((D, K), lambda i: (0, 0)),
        ],
        out_specs=pl.BlockSpec((BN, K), lambda i: (i, 0)),
        out_shape=jax.ShapeDtypeStruct((N, K), jnp.float32),
    )(x, centroids.T)
    assignments = _sc_argmin(scores)
    return (centroids[None, :, :], assignments)


# speedup vs baseline: 1.0261x; 1.0261x over previous
import functools

import jax
import jax.numpy as jnp
from jax import lax
from jax.experimental import pallas as pl
from jax.experimental.pallas import tpu as pltpu
from jax.experimental.pallas import tpu_sc as plsc

N, D, K = 8192, 32, 512
BN = 1024
NC, NS, L = 2, 16, 16
NW = NC * NS
PPW = N // NW
KB = K // L
CH = 32
NCH = PPW // CH

_mesh = plsc.VectorSubcoreMesh(core_axis_name="c", subcore_axis_name="s",
                               num_cores=NC, num_subcores=NS)


def _scores_body(x_ref, ct_ref, s_ref):
    x = x_ref[...]
    ct = ct_ref[...]
    cn = jnp.sum(ct * ct, axis=0)[None, :]
    dot = lax.dot_general(x, ct, (((1,), (0,)), ((), ())),
                          precision=lax.Precision.HIGHEST,
                          preferred_element_type=jnp.float32)
    s_ref[...] = cn - 2.0 * dot


@functools.partial(
    pl.kernel,
    out_type=jax.ShapeDtypeStruct((N,), jnp.int32),
    mesh=_mesh,
    scratch_types=[
        pltpu.VMEM((CH, K), jnp.float32),
        pltpu.VMEM((CH, K), jnp.float32),
        pltpu.VMEM((PPW,), jnp.int32),
        pltpu.SemaphoreType.DMA,
        pltpu.SemaphoreType.DMA,
    ],
    compiler_params=pltpu.CompilerParams(needs_layout_passes=False),
)
def _sc_argmin(s_hbm, out_hbm, buf0, buf1, out_v, sem0, sem1):
    wid = lax.axis_index("s") * NC + lax.axis_index("c")
    base = wid * PPW
    bufs = (buf0, buf1)
    sems = (sem0, sem1)
    iota = lax.iota(jnp.int32, L)
    lane0 = iota == 0

    cps = [None] * NCH
    cps[0] = pltpu.async_copy(s_hbm.at[pl.ds(base, CH)], buf0, sem0)
    for ch in range(NCH):
        if ch + 1 < NCH:
            cps[ch + 1] = pltpu.async_copy(
                s_hbm.at[pl.ds(base + (ch + 1) * CH, CH)],
                bufs[(ch + 1) % 2], sems[(ch + 1) % 2])
        cps[ch].wait()
        buf = bufs[ch % 2]
        rb = ch * CH

        def row_body(r, _, buf=buf, rb=rb):
            sv = [buf[r, pl.ds(kb * L, L)] for kb in range(KB)]
            bestv = sv[0]
            besti = jnp.zeros((L,), jnp.int32)
            for kb in range(1, KB):
                better = sv[kb] < bestv
                bestv = jnp.where(better, sv[kb], bestv)
                besti = jnp.where(better, jnp.full((L,), kb, jnp.int32),
                                  besti)
            m = jnp.min(bestv)
            cand = jnp.where(bestv == m, besti * L + iota, K)
            ridx = jnp.min(cand)
            plsc.store_scatter(out_v, [jnp.full((L,), rb, jnp.int32) + r],
                               jnp.full((L,), 0, jnp.int32) + ridx,
                               mask=lane0)
            return 0

        lax.fori_loop(0, CH, row_body, 0)
    pltpu.sync_copy(out_v, out_hbm.at[pl.ds(base, PPW)])


def kernel(x, centroids):
    scores = pl.pallas_call(
        _scores_body,
        grid=(N // BN,),
        in_specs=[
            pl.BlockSpec((BN, D), lambda i: (i, 0)),
            pl.BlockSpec((D, K), lambda i: (0, 0)),
        ],
        out_specs=pl.BlockSpec((BN, K), lambda i: (i, 0)),
        out_shape=jax.ShapeDtypeStruct((N, K), jnp.float32),
    )(x, centroids.T)
    assignments = _sc_argmin(scores)
    return (centroids[None, :, :], assignments)
